# baseline (speedup 1.0000x reference)
import functools

import jax
import jax.numpy as jnp
from jax import lax
from jax.experimental import pallas as pl
from jax.experimental.pallas import tpu as pltpu
from jax.experimental.pallas import tpu_sc as plsc

N = 10000
D = 128
E = 320000

NC = 2
NS = 16
K = 128
NCHUNK = 160
NCHUNKS_TOTAL = NS * NCHUNK
E_PAD = NCHUNKS_TOTAL * K
ZR = 80
NZC = N // ZR


def _mm_body(x_ref, w_ref, o_ref):
    r = jnp.dot(x_ref[...], w_ref[...], preferred_element_type=jnp.float32)
    r = r.reshape(400, D // 32, 2, 16).transpose(0, 1, 3, 2).reshape(400, D)
    o_ref[...] = r.astype(jnp.bfloat16)


_matmul = pl.pallas_call(
    _mm_body,
    grid=(25,),
    in_specs=[
        pl.BlockSpec((400, D), lambda i: (i, 0)),
        pl.BlockSpec((D, D), lambda i: (0, 0)),
    ],
    out_specs=pl.BlockSpec((400, D), lambda i: (i, 0)),
    out_shape=jax.ShapeDtypeStruct((N, D), jnp.bfloat16),
)


def _comb_body(p_ref, b_ref, o_ref):
    o_ref[...] = p_ref[...] + b_ref[...]


_combine = pl.pallas_call(
    _comb_body,
    grid=(25,),
    in_specs=[
        pl.BlockSpec((400, D), lambda i: (i, 0)),
        pl.BlockSpec((1, D), lambda i: (0, 0)),
    ],
    out_specs=pl.BlockSpec((400, D), lambda i: (i, 0)),
    out_shape=jax.ShapeDtypeStruct((N, D), jnp.float32),
)

_sc_mesh = plsc.VectorSubcoreMesh(
    core_axis_name="c", subcore_axis_name="s", num_cores=NC, num_subcores=NS)


@functools.partial(
    pl.kernel,
    out_type=jax.ShapeDtypeStruct((N, D), jnp.float32),
    mesh=_sc_mesh,
    compiler_params=pltpu.CompilerParams(needs_layout_passes=False, use_tc_tiling_on_sc=False),
    scratch_types=[
        pltpu.VMEM_SHARED((N, D), jnp.float32),
        [pltpu.VMEM((2, K), jnp.int32) for _ in range(4)],
        [pltpu.VMEM((K,), jnp.float32) for _ in range(4)],
        [pltpu.VMEM((K, D // 2), jnp.int32) for _ in range(2)],
        [pltpu.VMEM((K, D), jnp.float32) for _ in range(2)],
        [pltpu.SemaphoreType.DMA for _ in range(4)],
        [pltpu.SemaphoreType.DMA for _ in range(2)],
        [pltpu.SemaphoreType.DMA for _ in range(2)],
    ],
)
def _sc_aggregate(edata_hbm, vals_hbm, sup_hbm, out_hbm,
                  acc, eb, vb, rowsb, rows, esem, gsem, ssem):
    c = lax.axis_index("c")
    s = lax.axis_index("s")
    gbase = s * NCHUNK

    def _start_edata(j, q):
        pltpu.async_copy(edata_hbm.at[gbase + j], eb[q], esem[q])
        pltpu.async_copy(vals_hbm.at[gbase + j], vb[q], esem[q])

    def _wait_edata(q):
        pltpu.make_async_copy(edata_hbm.at[0], eb[q], esem[q]).wait()
        pltpu.make_async_copy(vals_hbm.at[0], vb[q], esem[q]).wait()

    def _start_gather(p, q):
        pltpu.async_copy(sup_hbm.at[eb[q].at[0]], rowsb[p], gsem[p])

    def _wait_gather(p):
        pltpu.make_async_copy(sup_hbm.at[eb[0].at[0]], rowsb[p],
                              gsem[p]).wait()

    def _start_scatter(p, q):
        pltpu.async_copy(rows[p], acc.at[eb[q].at[1]], ssem[p], add=True)

    def _wait_scatter(p):
        pltpu.make_async_copy(rows[p], acc.at[eb[0].at[1]],
                              ssem[p]).wait()

    def _scale(p, q):
        rb = rowsb[p]
        rp = rows[p]
        vq = vb[q]

        def _mul(g, inner):
            v16 = vq[pl.ds(g * 16, 16)]
            for i in range(16):
                vbc = jnp.full((16,), v16[i], jnp.float32)
                r = g * 16 + i
                for c2 in range(D // 32):
                    w = rb[r, pl.ds(c2 * 16, 16)]
                    a = plsc.bitcast(w << 16, jnp.float32)
                    b2 = plsc.bitcast(w & jnp.int32(-65536), jnp.float32)
                    rp[r, pl.ds(c2 * 32, 16)] = a * vbc
                    rp[r, pl.ds(c2 * 32 + 16, 16)] = b2 * vbc
            return inner

        lax.fori_loop(0, K // 16, _mul, 0)

    @pl.when(c == 0)
    def _run():
        zero16 = jnp.zeros((16,), jnp.float32)

        def _zrow(r, carry):
            for c8 in range(D // 16):
                rows[0][r, pl.ds(c8 * 16, 16)] = zero16
            return carry

        lax.fori_loop(0, ZR, _zrow, 0)
        for i in range(8):
            cid = s + i * NS
            @pl.when(cid < NZC)
            def _():
                pltpu.sync_copy(rows[0].at[pl.ds(0, ZR)],
                                acc.at[pl.ds(cid * ZR, ZR)])

        plsc.subcore_barrier()
        for q in range(3):
            _start_edata(q, q)
        _wait_edata(0)
        _start_gather(0, 0)

        def _quad(j4, carry):
            for k in range(4):
                j = 4 * j4 + k
                p = k % 2
                q = k
                _wait_gather(p)
                if k == 0:
                    @pl.when(j4 > 0)
                    def _():
                        _wait_scatter(1)
                else:
                    _wait_scatter(1 - p)
                @pl.when(j + 3 < NCHUNK)
                def _():
                    _start_edata(j + 3, (q + 3) % 4)
                @pl.when(j + 1 < NCHUNK)
                def _():
                    _wait_edata((q + 1) % 4)
                    _start_gather(1 - p, (q + 1) % 4)
                _scale(p, q)
                _start_scatter(p, q)
            return carry

        lax.fori_loop(0, NCHUNK // 4, _quad, 0)
        _wait_scatter(1)

        plsc.subcore_barrier()
        for i in range(8):
            cid = s + i * NS
            @pl.when(cid < NZC)
            def _():
                pltpu.sync_copy(acc.at[pl.ds(cid * ZR, ZR)],
                                out_hbm.at[pl.ds(cid * ZR, ZR)])


def kernel(input_feature, edge_index, adj_values, W, b):
    support = _matmul(input_feature, W)

    pad = E_PAD - E
    src = jnp.concatenate([edge_index[0], jnp.zeros((pad,), jnp.int32)])
    dst = jnp.concatenate([edge_index[1],
                           jnp.arange(pad, dtype=jnp.int32)])
    vals = jnp.concatenate([adj_values, jnp.zeros((pad,), jnp.float32)])
    edata = jnp.stack([src.reshape(NCHUNKS_TOTAL, K),
                       dst.reshape(NCHUNKS_TOTAL, K)], axis=1)

    sup_i32 = lax.bitcast_convert_type(
        support.reshape(N, D // 2, 2), jnp.int32)
    parts = _sc_aggregate(edata, vals.reshape(NCHUNKS_TOTAL, K), sup_i32)
    return _combine(parts, b.reshape(1, D))


# speedup vs baseline: 1.7172x; 1.7172x over previous
import functools

import jax
import jax.numpy as jnp
from jax import lax
from jax.experimental import pallas as pl
from jax.experimental.pallas import tpu as pltpu
from jax.experimental.pallas import tpu_sc as plsc

N = 10000
D = 128
E = 320000

NC = 2
NS = 16
K = 128
CH0 = 136
CH1 = 24
NCHUNKS_TOTAL = NS * (CH0 + CH1)
E_PAD = NCHUNKS_TOTAL * K
ZR = 80
NZC = N // ZR


def _mm_body(x_ref, w_ref, o_ref):
    o_ref[...] = jnp.dot(x_ref[...], w_ref[...],
                         preferred_element_type=jnp.float32)


_matmul = pl.pallas_call(
    _mm_body,
    grid=(25,),
    in_specs=[
        pl.BlockSpec((400, D), lambda i: (i, 0)),
        pl.BlockSpec((D, D), lambda i: (0, 0)),
    ],
    out_specs=pl.BlockSpec((400, D), lambda i: (i, 0)),
    out_shape=jax.ShapeDtypeStruct((N, D), jnp.float32),
)


def _comb_body(p0_ref, p1_ref, b_ref, o_ref):
    o_ref[...] = p0_ref[...] + p1_ref[...] + b_ref[...]


_combine = pl.pallas_call(
    _comb_body,
    grid=(25,),
    in_specs=[
        pl.BlockSpec((400, D), lambda i: (i, 0)),
        pl.BlockSpec((400, D), lambda i: (i + 25, 0)),
        pl.BlockSpec((1, D), lambda i: (0, 0)),
    ],
    out_specs=pl.BlockSpec((400, D), lambda i: (i, 0)),
    out_shape=jax.ShapeDtypeStruct((N, D), jnp.float32),
)

_sc_mesh = plsc.VectorSubcoreMesh(
    core_axis_name="c", subcore_axis_name="s", num_cores=NC, num_subcores=NS)


@functools.partial(
    pl.kernel,
    out_type=jax.ShapeDtypeStruct((NC * N, D), jnp.float32),
    mesh=_sc_mesh,
    scratch_types=[
        pltpu.VMEM_SHARED((N, D), jnp.float32),
        [pltpu.VMEM((2, K), jnp.int32) for _ in range(4)],
        [pltpu.VMEM((K,), jnp.float32) for _ in range(4)],
        [pltpu.VMEM((K, D), jnp.float32) for _ in range(2)],
        [pltpu.SemaphoreType.DMA for _ in range(4)],
        [pltpu.SemaphoreType.DMA for _ in range(2)],
        [pltpu.SemaphoreType.DMA for _ in range(2)],
    ],
)
def _sc_aggregate(edata_hbm, vals_hbm, sup_hbm, out_hbm,
                  acc, eb, vb, rows, esem, gsem, ssem):
    c = lax.axis_index("c")
    s = lax.axis_index("s")
    gbase = jnp.where(c == 0, s * CH0, NS * CH0 + s * CH1)
    nchunk = jnp.where(c == 0, CH0, CH1)

    zero16 = jnp.zeros((16,), jnp.float32)

    def _zrow(r, carry):
        for c8 in range(D // 16):
            rows[0][r, pl.ds(c8 * 16, 16)] = zero16
        return carry

    lax.fori_loop(0, ZR, _zrow, 0)
    for i in range(8):
        cid = s + i * NS
        @pl.when(cid < NZC)
        def _():
            pltpu.sync_copy(rows[0].at[pl.ds(0, ZR)],
                            acc.at[pl.ds(cid * ZR, ZR)])

    def _start_edata(j, q):
        pltpu.async_copy(edata_hbm.at[gbase + j], eb[q], esem[q])
        pltpu.async_copy(vals_hbm.at[gbase + j], vb[q], esem[q])

    def _wait_edata(q):
        pltpu.make_async_copy(edata_hbm.at[0], eb[q], esem[q]).wait()
        pltpu.make_async_copy(vals_hbm.at[0], vb[q], esem[q]).wait()

    def _start_gather(p, q):
        pltpu.async_copy(sup_hbm.at[eb[q].at[0]], rows[p], gsem[p])

    def _wait_gather(p):
        pltpu.make_async_copy(sup_hbm.at[eb[0].at[0]], rows[p],
                              gsem[p]).wait()

    def _start_scatter(p, q):
        pltpu.async_copy(rows[p], acc.at[eb[q].at[1]], ssem[p], add=True)

    def _wait_scatter(p):
        pltpu.make_async_copy(rows[p], acc.at[eb[0].at[1]],
                              ssem[p]).wait()

    def _scale(p, q):
        rp = rows[p]
        vq = vb[q]

        def _mul(g, inner):
            v16 = vq[pl.ds(g * 16, 16)]
            for i in range(16):
                vbc = jnp.full((16,), v16[i], jnp.float32)
                r = g * 16 + i
                for c8 in range(D // 16):
                    sl = pl.ds(c8 * 16, 16)
                    rp[r, sl] = rp[r, sl] * vbc
            return inner

        lax.fori_loop(0, K // 16, _mul, 0)

    plsc.subcore_barrier()
    for q in range(3):
        _start_edata(q, q)
    _wait_edata(0)
    _start_gather(0, 0)

    def _quad(j4, carry):
        for k in range(4):
            j = 4 * j4 + k
            p = k % 2
            q = k
            _wait_gather(p)
            if k == 0:
                @pl.when(j4 > 0)
                def _():
                    _wait_scatter(1)
            else:
                _wait_scatter(1 - p)
            @pl.when(j + 3 < nchunk)
            def _():
                _start_edata(j + 3, (q + 3) % 4)
            @pl.when(j + 1 < nchunk)
            def _():
                _wait_edata((q + 1) % 4)
                _start_gather(1 - p, (q + 1) % 4)
            _scale(p, q)
            _start_scatter(p, q)
        return carry

    lax.fori_loop(0, nchunk // 4, _quad, 0)
    _wait_scatter(1)

    plsc.subcore_barrier()
    for i in range(8):
        cid = s + i * NS
        @pl.when(cid < NZC)
        def _():
            pltpu.sync_copy(acc.at[pl.ds(cid * ZR, ZR)],
                            out_hbm.at[pl.ds(c * N + cid * ZR, ZR)])


def kernel(input_feature, edge_index, adj_values, W, b):
    support = _matmul(input_feature, W)

    pad = E_PAD - E
    src = jnp.concatenate([edge_index[0], jnp.zeros((pad,), jnp.int32)])
    dst = jnp.concatenate([edge_index[1],
                           jnp.arange(pad, dtype=jnp.int32)])
    vals = jnp.concatenate([adj_values, jnp.zeros((pad,), jnp.float32)])
    edata = jnp.stack([src.reshape(NCHUNKS_TOTAL, K),
                       dst.reshape(NCHUNKS_TOTAL, K)], axis=1)

    parts = _sc_aggregate(edata, vals.reshape(NCHUNKS_TOTAL, K), support)
    return _combine(parts, parts, b.reshape(1, D))


# speedup vs baseline: 1.7188x; 1.0009x over previous
import functools

import jax
import jax.numpy as jnp
from jax import lax
from jax.experimental import pallas as pl
from jax.experimental.pallas import tpu as pltpu
from jax.experimental.pallas import tpu_sc as plsc

N = 10000
D = 128
E = 320000

NC = 2
NS = 16
K = 128
CH0 = 136
CH1 = 24
NCHUNKS_TOTAL = NS * (CH0 + CH1)
E_PAD = NCHUNKS_TOTAL * K
ZR = 80
NZC = N // ZR


def _mm_body(x_ref, w_ref, o_ref):
    o_ref[...] = jnp.dot(x_ref[...], w_ref[...],
                         preferred_element_type=jnp.float32)


_matmul = pl.pallas_call(
    _mm_body,
    grid=(25,),
    in_specs=[
        pl.BlockSpec((400, D), lambda i: (i, 0)),
        pl.BlockSpec((D, D), lambda i: (0, 0)),
    ],
    out_specs=pl.BlockSpec((400, D), lambda i: (i, 0)),
    out_shape=jax.ShapeDtypeStruct((N, D), jnp.float32),
)


def _comb_body(p0_ref, p1_ref, b_ref, o_ref):
    o_ref[...] = p0_ref[...] + p1_ref[...] + b_ref[...]


_combine = pl.pallas_call(
    _comb_body,
    grid=(25,),
    in_specs=[
        pl.BlockSpec((400, D), lambda i: (i, 0)),
        pl.BlockSpec((400, D), lambda i: (i + 25, 0)),
        pl.BlockSpec((1, D), lambda i: (0, 0)),
    ],
    out_specs=pl.BlockSpec((400, D), lambda i: (i, 0)),
    out_shape=jax.ShapeDtypeStruct((N, D), jnp.float32),
)

_sc_mesh = plsc.VectorSubcoreMesh(
    core_axis_name="c", subcore_axis_name="s", num_cores=NC, num_subcores=NS)


@functools.partial(
    pl.kernel,
    out_type=jax.ShapeDtypeStruct((NC * N, D), jnp.float32),
    mesh=_sc_mesh,
    scratch_types=[
        pltpu.VMEM_SHARED((N, D), jnp.float32),
        [pltpu.VMEM((2, K), jnp.int32) for _ in range(4)],
        [pltpu.VMEM((K,), jnp.float32) for _ in range(4)],
        [pltpu.VMEM((K, D), jnp.float32) for _ in range(2)],
        [pltpu.SemaphoreType.DMA for _ in range(4)],
        [pltpu.SemaphoreType.DMA for _ in range(2)],
        [pltpu.SemaphoreType.DMA for _ in range(2)],
    ],
)
def _sc_aggregate(edata_hbm, vals_hbm, sup_hbm, out_hbm,
                  acc, eb, vb, rows, esem, gsem, ssem):
    c = lax.axis_index("c")
    s = lax.axis_index("s")
    gbase = jnp.where(c == 0, s * CH0, NS * CH0 + s * CH1)
    nchunk = jnp.where(c == 0, CH0, CH1)

    zero16 = jnp.zeros((16,), jnp.float32)

    def _zrow(r, carry):
        for c8 in range(D // 16):
            rows[0][r, pl.ds(c8 * 16, 16)] = zero16
        return carry

    lax.fori_loop(0, ZR, _zrow, 0)
    for i in range(8):
        cid = s + i * NS
        @pl.when(cid < NZC)
        def _():
            pltpu.async_copy(rows[0].at[pl.ds(0, ZR)],
                             acc.at[pl.ds(cid * ZR, ZR)], gsem[0])
    for i in range(8):
        cid = s + i * NS
        @pl.when(cid < NZC)
        def _():
            pltpu.make_async_copy(rows[0].at[pl.ds(0, ZR)],
                                  acc.at[pl.ds(cid * ZR, ZR)],
                                  gsem[0]).wait()

    def _start_edata(j, q):
        pltpu.async_copy(edata_hbm.at[gbase + j], eb[q], esem[q])
        pltpu.async_copy(vals_hbm.at[gbase + j], vb[q], esem[q])

    def _wait_edata(q):
        pltpu.make_async_copy(edata_hbm.at[0], eb[q], esem[q]).wait()
        pltpu.make_async_copy(vals_hbm.at[0], vb[q], esem[q]).wait()

    def _start_gather(p, q):
        pltpu.async_copy(sup_hbm.at[eb[q].at[0]], rows[p], gsem[p])

    def _wait_gather(p):
        pltpu.make_async_copy(sup_hbm.at[eb[0].at[0]], rows[p],
                              gsem[p]).wait()

    def _start_scatter(p, q):
        pltpu.async_copy(rows[p], acc.at[eb[q].at[1]], ssem[p], add=True)

    def _wait_scatter(p):
        pltpu.make_async_copy(rows[p], acc.at[eb[0].at[1]],
                              ssem[p]).wait()

    def _scale(p, q):
        rp = rows[p]
        vq = vb[q]

        def _mul(g, inner):
            v16 = vq[pl.ds(g * 16, 16)]
            for i in range(16):
                vbc = jnp.full((16,), v16[i], jnp.float32)
                r = g * 16 + i
                for c8 in range(D // 16):
                    sl = pl.ds(c8 * 16, 16)
                    rp[r, sl] = rp[r, sl] * vbc
            return inner

        lax.fori_loop(0, K // 16, _mul, 0)

    plsc.subcore_barrier()
    for q in range(3):
        _start_edata(q, q)
    _wait_edata(0)
    _start_gather(0, 0)

    def _quad(j4, carry):
        for k in range(4):
            j = 4 * j4 + k
            p = k % 2
            q = k
            _wait_gather(p)
            if k == 0:
                @pl.when(j4 > 0)
                def _():
                    _wait_scatter(1)
            else:
                _wait_scatter(1 - p)
            @pl.when(j + 3 < nchunk)
            def _():
                _start_edata(j + 3, (q + 3) % 4)
            @pl.when(j + 1 < nchunk)
            def _():
                _wait_edata((q + 1) % 4)
                _start_gather(1 - p, (q + 1) % 4)
            _scale(p, q)
            _start_scatter(p, q)
        return carry

    lax.fori_loop(0, nchunk // 4, _quad, 0)
    _wait_scatter(1)

    plsc.subcore_barrier()
    for i in range(8):
        cid = s + i * NS
        @pl.when(cid < NZC)
        def _():
            pltpu.async_copy(acc.at[pl.ds(cid * ZR, ZR)],
                             out_hbm.at[pl.ds(c * N + cid * ZR, ZR)],
                             ssem[0])
    for i in range(8):
        cid = s + i * NS
        @pl.when(cid < NZC)
        def _():
            pltpu.make_async_copy(acc.at[pl.ds(cid * ZR, ZR)],
                                  out_hbm.at[pl.ds(c * N + cid * ZR, ZR)],
                                  ssem[0]).wait()


def kernel(input_feature, edge_index, adj_values, W, b):
    support = _matmul(input_feature, W)

    pad = E_PAD - E
    src = jnp.concatenate([edge_index[0], jnp.zeros((pad,), jnp.int32)])
    dst = jnp.concatenate([edge_index[1],
                           jnp.arange(pad, dtype=jnp.int32)])
    vals = jnp.concatenate([adj_values, jnp.zeros((pad,), jnp.float32)])
    edata = jnp.stack([src.reshape(NCHUNKS_TOTAL, K),
                       dst.reshape(NCHUNKS_TOTAL, K)], axis=1)

    parts = _sc_aggregate(edata, vals.reshape(NCHUNKS_TOTAL, K), support)
    return _combine(parts, parts, b.reshape(1, D))


# speedup vs baseline: 1.8910x; 1.1002x over previous
import functools

import jax
import jax.numpy as jnp
from jax import lax
from jax.experimental import pallas as pl
from jax.experimental.pallas import tpu as pltpu
from jax.experimental.pallas import tpu_sc as plsc

N = 10000
D = 128
E = 320000

NC = 2
NS = 16
K = 128
CH0 = 148
CH1 = 12
NCHUNKS_TOTAL = NS * (CH0 + CH1)
E_PAD = NCHUNKS_TOTAL * K
ZR = 80
NZC = N // ZR


def _mm_body(x_ref, w_ref, o_ref):
    o_ref[...] = jnp.dot(x_ref[...], w_ref[...],
                         preferred_element_type=jnp.float32)


_matmul = pl.pallas_call(
    _mm_body,
    grid=(25,),
    in_specs=[
        pl.BlockSpec((400, D), lambda i: (i, 0)),
        pl.BlockSpec((D, D), lambda i: (0, 0)),
    ],
    out_specs=pl.BlockSpec((400, D), lambda i: (i, 0)),
    out_shape=jax.ShapeDtypeStruct((N, D), jnp.float32),
)


def _comb_body(p0_ref, p1_ref, b_ref, o_ref):
    o_ref[...] = p0_ref[...] + p1_ref[...] + b_ref[...]


_combine = pl.pallas_call(
    _comb_body,
    grid=(25,),
    in_specs=[
        pl.BlockSpec((400, D), lambda i: (i, 0)),
        pl.BlockSpec((400, D), lambda i: (i + 25, 0)),
        pl.BlockSpec((1, D), lambda i: (0, 0)),
    ],
    out_specs=pl.BlockSpec((400, D), lambda i: (i, 0)),
    out_shape=jax.ShapeDtypeStruct((N, D), jnp.float32),
)

_sc_mesh = plsc.VectorSubcoreMesh(
    core_axis_name="c", subcore_axis_name="s", num_cores=NC, num_subcores=NS)


@functools.partial(
    pl.kernel,
    out_type=jax.ShapeDtypeStruct((NC * N, D), jnp.float32),
    mesh=_sc_mesh,
    scratch_types=[
        pltpu.VMEM_SHARED((N, D), jnp.float32),
        [pltpu.VMEM((2, K), jnp.int32) for _ in range(4)],
        [pltpu.VMEM((K,), jnp.float32) for _ in range(4)],
        [pltpu.VMEM((K, D), jnp.float32) for _ in range(2)],
        [pltpu.SemaphoreType.DMA for _ in range(4)],
        [pltpu.SemaphoreType.DMA for _ in range(2)],
        [pltpu.SemaphoreType.DMA for _ in range(2)],
    ],
)
def _sc_aggregate(edata_hbm, vals_hbm, sup_hbm, out_hbm,
                  acc, eb, vb, rows, esem, gsem, ssem):
    c = lax.axis_index("c")
    s = lax.axis_index("s")
    gbase = jnp.where(c == 0, s * CH0, NS * CH0 + s * CH1)
    nchunk = jnp.where(c == 0, CH0, CH1)

    zero16 = jnp.zeros((16,), jnp.float32)

    def _zrow(r, carry):
        for c8 in range(D // 16):
            rows[0][r, pl.ds(c8 * 16, 16)] = zero16
        return carry

    lax.fori_loop(0, ZR, _zrow, 0)
    for i in range(8):
        cid = s + i * NS
        @pl.when(cid < NZC)
        def _():
            pltpu.async_copy(rows[0].at[pl.ds(0, ZR)],
                             acc.at[pl.ds(cid * ZR, ZR)], gsem[0])
    for i in range(8):
        cid = s + i * NS
        @pl.when(cid < NZC)
        def _():
            pltpu.make_async_copy(rows[0].at[pl.ds(0, ZR)],
                                  acc.at[pl.ds(cid * ZR, ZR)],
                                  gsem[0]).wait()

    def _start_edata(j, q):
        pltpu.async_copy(edata_hbm.at[gbase + j], eb[q], esem[q])
        pltpu.async_copy(vals_hbm.at[gbase + j], vb[q], esem[q])

    def _wait_edata(q):
        pltpu.make_async_copy(edata_hbm.at[0], eb[q], esem[q]).wait()
        pltpu.make_async_copy(vals_hbm.at[0], vb[q], esem[q]).wait()

    def _start_gather(p, q):
        pltpu.async_copy(sup_hbm.at[eb[q].at[0]], rows[p], gsem[p])

    def _wait_gather(p):
        pltpu.make_async_copy(sup_hbm.at[eb[0].at[0]], rows[p],
                              gsem[p]).wait()

    def _start_scatter(p, q):
        pltpu.async_copy(rows[p], acc.at[eb[q].at[1]], ssem[p], add=True)

    def _wait_scatter(p):
        pltpu.make_async_copy(rows[p], acc.at[eb[0].at[1]],
                              ssem[p]).wait()

    def _scale(p, q):
        rp = rows[p]
        vq = vb[q]

        def _mul(g, inner):
            v16 = vq[pl.ds(g * 16, 16)]
            for i in range(16):
                vbc = jnp.full((16,), v16[i], jnp.float32)
                r = g * 16 + i
                for c8 in range(D // 16):
                    sl = pl.ds(c8 * 16, 16)
                    rp[r, sl] = rp[r, sl] * vbc
            return inner

        lax.fori_loop(0, K // 16, _mul, 0)

    plsc.subcore_barrier()
    for q in range(3):
        _start_edata(q, q)
    _wait_edata(0)
    _start_gather(0, 0)

    def _quad(j4, carry):
        for k in range(4):
            j = 4 * j4 + k
            p = k % 2
            q = k
            _wait_gather(p)
            if k == 0:
                @pl.when(j4 > 0)
                def _():
                    _wait_scatter(1)
            else:
                _wait_scatter(1 - p)
            @pl.when(j + 3 < nchunk)
            def _():
                _start_edata(j + 3, (q + 3) % 4)
            @pl.when(j + 1 < nchunk)
            def _():
                _wait_edata((q + 1) % 4)
                _start_gather(1 - p, (q + 1) % 4)
            _scale(p, q)
            _start_scatter(p, q)
        return carry

    lax.fori_loop(0, nchunk // 4, _quad, 0)
    _wait_scatter(1)

    plsc.subcore_barrier()
    for i in range(8):
        cid = s + i * NS
        @pl.when(cid < NZC)
        def _():
            pltpu.async_copy(acc.at[pl.ds(cid * ZR, ZR)],
                             out_hbm.at[pl.ds(c * N + cid * ZR, ZR)],
                             ssem[0])
    for i in range(8):
        cid = s + i * NS
        @pl.when(cid < NZC)
        def _():
            pltpu.make_async_copy(acc.at[pl.ds(cid * ZR, ZR)],
                                  out_hbm.at[pl.ds(c * N + cid * ZR, ZR)],
                                  ssem[0]).wait()


def kernel(input_feature, edge_index, adj_values, W, b):
    support = _matmul(input_feature, W)

    pad = E_PAD - E
    src = jnp.concatenate([edge_index[0], jnp.zeros((pad,), jnp.int32)])
    dst = jnp.concatenate([edge_index[1],
                           jnp.arange(pad, dtype=jnp.int32)])
    vals = jnp.concatenate([adj_values, jnp.zeros((pad,), jnp.float32)])
    edata = jnp.stack([src.reshape(NCHUNKS_TOTAL, K),
                       dst.reshape(NCHUNKS_TOTAL, K)], axis=1)

    parts = _sc_aggregate(edata, vals.reshape(NCHUNKS_TOTAL, K), support)
    return _combine(parts, parts, b.reshape(1, D))
